# baseline (speedup 1.0000x reference)
import functools

import jax
import jax.numpy as jnp
from jax import lax
from jax.experimental import pallas as pl
from jax.experimental.pallas import tpu as pltpu
from jax.experimental.pallas import tpu_sc as plsc

N = 10000
E = 320000
D = 128
DE = 16
MSG = 128

NC = 2
NS = 16
NW = NC * NS

CHUNK = 128
NCHUNK = E // CHUNK
ITERS = (NCHUNK + NW - 1) // NW
EPAD = NW * ITERS * CHUNK

NPAD = 10240
ROWS_PER_TILE = NPAD // NS

_sc_mesh = lambda: plsc.VectorSubcoreMesh(core_axis_name="c", subcore_axis_name="s")


def _make_diff_gather():
    @functools.partial(
        pl.kernel,
        out_type=jax.ShapeDtypeStruct((E, D), jnp.float32),
        mesh=_sc_mesh(),
        scratch_types=[
            pltpu.VMEM((ITERS, CHUNK), jnp.int32),
            pltpu.VMEM((ITERS, CHUNK), jnp.int32),
            [pltpu.VMEM((CHUNK, D), jnp.float32)] * 2,
            [pltpu.VMEM((CHUNK, D), jnp.float32)] * 2,
            [pltpu.VMEM((CHUNK, D), jnp.float32)] * 2,
            [pltpu.SemaphoreType.DMA] * 2,
            [pltpu.SemaphoreType.DMA] * 2,
            [pltpu.SemaphoreType.DMA] * 2,
        ],
    )
    def diff_gather(state_hbm, src_hbm, dst_hbm, out_hbm,
                    sidx, didx, rows_a, rows_b, rows_o,
                    sem_a, sem_b, sem_o):
        c = lax.axis_index("c")
        s = lax.axis_index("s")
        wid = s * NC + c

        pltpu.sync_copy(src_hbm.at[wid], sidx)
        pltpu.sync_copy(dst_hbm.at[wid], didx)

        def start(i, b):
            chunk = wid * ITERS + i

            @pl.when(jnp.logical_and(i < ITERS, chunk < NCHUNK))
            def _():
                pltpu.async_copy(state_hbm.at[sidx.at[i]], rows_a[b],
                                 sem_a[b])
                pltpu.async_copy(state_hbm.at[didx.at[i]], rows_b[b],
                                 sem_b[b])

        def finish(i, b):
            chunk = wid * ITERS + i

            @pl.when(jnp.logical_and(i < ITERS, chunk < NCHUNK))
            def _():
                base = chunk * CHUNK
                pltpu.make_async_copy(state_hbm.at[sidx.at[i]], rows_a[b],
                                      sem_a[b]).wait()
                pltpu.make_async_copy(state_hbm.at[didx.at[i]], rows_b[b],
                                      sem_b[b]).wait()

                @pl.when(i >= 2)
                def _wait_prev_store():
                    pltpu.make_async_copy(
                        rows_o[b], out_hbm.at[pl.ds(0, CHUNK)], sem_o[b]
                    ).wait()

                def sub_row(r, carry2):
                    def sub_vec(j, carry3):
                        off = j * 16
                        rows_o[b][r, pl.ds(off, 16)] = (
                            rows_a[b][r, pl.ds(off, 16)]
                            - rows_b[b][r, pl.ds(off, 16)]
                        )
                        return carry3
                    return lax.fori_loop(0, D // 16, sub_vec, carry2,
                                         unroll=8)

                lax.fori_loop(0, CHUNK, sub_row, 0)
                pltpu.async_copy(rows_o[b], out_hbm.at[pl.ds(base, CHUNK)],
                                 sem_o[b])

        start(0, 0)

        @pl.loop(0, ITERS + 1, step=2)
        def _loop(i0):
            start(i0 + 1, 1)
            finish(i0, 0)
            start(i0 + 2, 0)
            finish(i0 + 1, 1)

        for b in range(2):
            pltpu.make_async_copy(rows_o[b], out_hbm.at[pl.ds(0, CHUNK)],
                                  sem_o[b]).wait()

    return diff_gather


def _make_scatter_add():
    @functools.partial(
        pl.kernel,
        out_type=jax.ShapeDtypeStruct((NC, NPAD, D), jnp.float32),
        mesh=_sc_mesh(),
        scratch_types=[
            [pltpu.VMEM((CHUNK,), jnp.int32)] * 2,
            [pltpu.VMEM((CHUNK, D), jnp.float32)] * 2,
            [pltpu.SemaphoreType.DMA] * 2,
            [pltpu.SemaphoreType.DMA] * 2,
            pltpu.VMEM_SHARED((NPAD, D), jnp.float32),
        ],
    )
    def scatter_add(msg_hbm, dst_hbm, zeros_hbm, out_hbm,
                    didx, rows, sem_i, sem_m, acc):
        c = lax.axis_index("c")
        s = lax.axis_index("s")
        wid = s * NC + c
        slab = s * ROWS_PER_TILE

        pltpu.sync_copy(zeros_hbm, acc.at[pl.ds(slab, ROWS_PER_TILE)])
        plsc.subcore_barrier()

        def start(i, b):
            chunk = wid + i * NW

            @pl.when(chunk < NCHUNK)
            def _():
                base = chunk * CHUNK
                pltpu.async_copy(dst_hbm.at[pl.ds(base, CHUNK)], didx[b],
                                 sem_i[b])
                pltpu.async_copy(msg_hbm.at[pl.ds(base, CHUNK)], rows[b],
                                 sem_m[b])

        def finish(i, b):
            chunk = wid + i * NW

            @pl.when(chunk < NCHUNK)
            def _():
                base = chunk * CHUNK
                pltpu.make_async_copy(dst_hbm.at[pl.ds(base, CHUNK)], didx[b],
                                      sem_i[b]).wait()
                pltpu.make_async_copy(msg_hbm.at[pl.ds(base, CHUNK)], rows[b],
                                      sem_m[b]).wait()
                pltpu.sync_copy(rows[b], acc.at[didx[b]], add=True)

        start(0, 0)

        @pl.loop(0, ITERS + 1, step=2)
        def _loop(i0):
            start(i0 + 1, 1)
            finish(i0, 0)
            start(i0 + 2, 0)
            finish(i0 + 1, 1)

        plsc.subcore_barrier()
        pltpu.sync_copy(acc.at[pl.ds(slab, ROWS_PER_TILE)],
                        out_hbm.at[c, pl.ds(slab, ROWS_PER_TILE)])

    return scatter_add


def _mlp_body(diff_ref, ef_ref, w1dT_ref, w1eT_ref, b1_ref,
              w2mT_ref, b2m_ref, w2aT_ref, b2a_ref, out_ref):
    x = diff_ref[...].astype(jnp.bfloat16)
    ef = ef_ref[...]
    h = (jnp.dot(x, w1dT_ref[...], preferred_element_type=jnp.float32)
         + jnp.dot(ef, w1eT_ref[...], preferred_element_type=jnp.float32)
         + b1_ref[...])
    h = jnp.maximum(h, 0.0).astype(jnp.bfloat16)
    hm = h[:, :MSG]
    ha = h[:, MSG:]
    msg = jnp.dot(hm, w2mT_ref[...], preferred_element_type=jnp.float32) + b2m_ref[...]
    att = jax.nn.sigmoid(
        jnp.dot(ha, w2aT_ref[...], preferred_element_type=jnp.float32)
        + b2a_ref[...])
    out_ref[...] = msg * att


def _run_mlp(diff, edge_feat, w1dT, w1eT, b1, w2mT, b2m, w2aT, b2a):
    BE = 2560
    grid = (E // BE,)
    full = lambda shape: pl.BlockSpec(shape, lambda i: (0, 0))
    return pl.pallas_call(
        _mlp_body,
        grid=grid,
        in_specs=[
            pl.BlockSpec((BE, D), lambda i: (i, 0)),
            pl.BlockSpec((BE, DE), lambda i: (i, 0)),
            full((D, 2 * MSG)),
            full((DE, 2 * MSG)),
            full((1, 2 * MSG)),
            full((MSG, MSG)),
            full((1, MSG)),
            full((MSG, MSG)),
            full((1, MSG)),
        ],
        out_specs=pl.BlockSpec((BE, MSG), lambda i: (i, 0)),
        out_shape=jax.ShapeDtypeStruct((E, MSG), jnp.float32),
        compiler_params=pltpu.CompilerParams(
            dimension_semantics=("arbitrary",),
        ),
    )(diff, edge_feat, w1dT, w1eT, b1, w2mT, b2m, w2aT, b2a)


def _gru_body(p0_ref, p1_ref, state_ref, wihT_ref, whhT_ref, bih_ref, bhh_ref,
              out_ref):
    m = p0_ref[...] + p1_ref[...]
    state = state_ref[...]
    gi = m @ wihT_ref[...] + bih_ref[...]
    gh = state @ whhT_ref[...] + bhh_ref[...]
    r = jax.nn.sigmoid(gi[:, :D] + gh[:, :D])
    z = jax.nn.sigmoid(gi[:, D:2 * D] + gh[:, D:2 * D])
    n = jnp.tanh(gi[:, 2 * D:] + r * gh[:, 2 * D:])
    out_ref[...] = (1.0 - z) * n + z * state


def _run_gru(p0, p1, state, wihT, whhT, bih, bhh):
    BN = 1000
    grid = (N // BN,)
    full = lambda shape: pl.BlockSpec(shape, lambda i: (0, 0))
    return pl.pallas_call(
        _gru_body,
        grid=grid,
        in_specs=[
            pl.BlockSpec((BN, MSG), lambda i: (i, 0)),
            pl.BlockSpec((BN, MSG), lambda i: (i, 0)),
            pl.BlockSpec((BN, D), lambda i: (i, 0)),
            full((MSG, 3 * D)),
            full((D, 3 * D)),
            full((1, 3 * D)),
            full((1, 3 * D)),
        ],
        out_specs=pl.BlockSpec((BN, D), lambda i: (i, 0)),
        out_shape=jax.ShapeDtypeStruct((N, D), jnp.float32),
        compiler_params=pltpu.CompilerParams(
            dimension_semantics=("arbitrary",),
        ),
    )(p0, p1, state, wihT, whhT, bih, bhh)


def kernel(node_feat, edge, edge_feat, msg_W1, msg_b1, msg_W2, msg_b2,
           att_W1, att_b1, att_W2, att_b2, gru_Wih, gru_Whh, gru_bih, gru_bhh):
    src = edge[:, 0].astype(jnp.int32)
    dst = edge[:, 1].astype(jnp.int32)
    state = node_feat.astype(jnp.float32)

    src3d = jnp.pad(src, (0, EPAD - E)).reshape(NW, ITERS, CHUNK)
    dst3d = jnp.pad(dst, (0, EPAD - E)).reshape(NW, ITERS, CHUNK)
    diff = _make_diff_gather()(state, src3d, dst3d)

    w1dT = jnp.concatenate([msg_W1[:, :D].T, att_W1[:, :D].T], axis=1)
    w1eT = jnp.concatenate([msg_W1[:, D:].T, att_W1[:, D:].T], axis=1)
    b1 = jnp.concatenate([msg_b1, att_b1]).reshape(1, 2 * MSG)
    msg = _run_mlp(diff, edge_feat.astype(jnp.bfloat16),
                   w1dT.astype(jnp.bfloat16), w1eT.astype(jnp.bfloat16), b1,
                   msg_W2.T.astype(jnp.bfloat16), msg_b2.reshape(1, MSG),
                   att_W2.T.astype(jnp.bfloat16), att_b2.reshape(1, MSG))

    zeros = jnp.zeros((ROWS_PER_TILE, D), jnp.float32)
    partials = _make_scatter_add()(msg, dst, zeros)

    return _run_gru(partials[0, :N], partials[1, :N], state,
                    gru_Wih.T, gru_Whh.T,
                    gru_bih.reshape(1, 3 * D), gru_bhh.reshape(1, 3 * D))


# speedup vs baseline: 1.0430x; 1.0430x over previous
import functools

import jax
import jax.numpy as jnp
from jax import lax
from jax.experimental import pallas as pl
from jax.experimental.pallas import tpu as pltpu
from jax.experimental.pallas import tpu_sc as plsc

N = 10000
E = 320000
D = 128
DE = 16
MSG = 128

NC = 2
NS = 16
NW = NC * NS

CHUNK = 128
EH = E // 2
NCHUNK = EH // CHUNK
ITERS = (NCHUNK + NW - 1) // NW
EPAD = NW * ITERS * CHUNK

NPAD = 10240
ROWS_PER_TILE = NPAD // NS

_sc_mesh = lambda: plsc.VectorSubcoreMesh(core_axis_name="c", subcore_axis_name="s")


def _make_diff_gather():
    @functools.partial(
        pl.kernel,
        out_type=jax.ShapeDtypeStruct((EH, D), jnp.float32),
        mesh=_sc_mesh(),
        scratch_types=[
            pltpu.VMEM((ITERS, CHUNK), jnp.int32),
            pltpu.VMEM((ITERS, CHUNK), jnp.int32),
            [pltpu.VMEM((CHUNK, D), jnp.float32)] * 2,
            [pltpu.VMEM((CHUNK, D), jnp.float32)] * 2,
            [pltpu.VMEM((CHUNK, D), jnp.float32)] * 2,
            [pltpu.SemaphoreType.DMA] * 2,
            [pltpu.SemaphoreType.DMA] * 2,
            [pltpu.SemaphoreType.DMA] * 2,
        ],
    )
    def diff_gather(state_hbm, src_hbm, dst_hbm, out_hbm,
                    sidx, didx, rows_a, rows_b, rows_o,
                    sem_a, sem_b, sem_o):
        c = lax.axis_index("c")
        s = lax.axis_index("s")
        wid = s * NC + c

        pltpu.sync_copy(src_hbm.at[wid], sidx)
        pltpu.sync_copy(dst_hbm.at[wid], didx)

        def start(i, b):
            chunk = wid * ITERS + i

            @pl.when(jnp.logical_and(i < ITERS, chunk < NCHUNK))
            def _():
                pltpu.async_copy(state_hbm.at[sidx.at[i]], rows_a[b],
                                 sem_a[b])
                pltpu.async_copy(state_hbm.at[didx.at[i]], rows_b[b],
                                 sem_b[b])

        def finish(i, b):
            chunk = wid * ITERS + i

            @pl.when(jnp.logical_and(i < ITERS, chunk < NCHUNK))
            def _():
                base = chunk * CHUNK
                pltpu.make_async_copy(state_hbm.at[sidx.at[i]], rows_a[b],
                                      sem_a[b]).wait()
                pltpu.make_async_copy(state_hbm.at[didx.at[i]], rows_b[b],
                                      sem_b[b]).wait()

                @pl.when(i >= 2)
                def _wait_prev_store():
                    pltpu.make_async_copy(
                        rows_o[b], out_hbm.at[pl.ds(0, CHUNK)], sem_o[b]
                    ).wait()

                def sub_row(r, carry2):
                    def sub_vec(j, carry3):
                        off = j * 16
                        rows_o[b][r, pl.ds(off, 16)] = (
                            rows_a[b][r, pl.ds(off, 16)]
                            - rows_b[b][r, pl.ds(off, 16)]
                        )
                        return carry3
                    return lax.fori_loop(0, D // 16, sub_vec, carry2,
                                         unroll=8)

                lax.fori_loop(0, CHUNK, sub_row, 0)
                pltpu.async_copy(rows_o[b], out_hbm.at[pl.ds(base, CHUNK)],
                                 sem_o[b])

        start(0, 0)

        @pl.loop(0, ITERS + 1, step=2)
        def _loop(i0):
            start(i0 + 1, 1)
            finish(i0, 0)
            start(i0 + 2, 0)
            finish(i0 + 1, 1)

        for b in range(2):
            pltpu.make_async_copy(rows_o[b], out_hbm.at[pl.ds(0, CHUNK)],
                                  sem_o[b]).wait()

    return diff_gather


def _make_scatter_add():
    @functools.partial(
        pl.kernel,
        out_type=jax.ShapeDtypeStruct((NC, NPAD, D), jnp.float32),
        mesh=_sc_mesh(),
        scratch_types=[
            [pltpu.VMEM((CHUNK,), jnp.int32)] * 2,
            [pltpu.VMEM((CHUNK, D), jnp.float32)] * 2,
            [pltpu.SemaphoreType.DMA] * 2,
            [pltpu.SemaphoreType.DMA] * 2,
            pltpu.VMEM_SHARED((NPAD, D), jnp.float32),
        ],
    )
    def scatter_add(msg_hbm, dst_hbm, zeros_hbm, out_hbm,
                    didx, rows, sem_i, sem_m, acc):
        c = lax.axis_index("c")
        s = lax.axis_index("s")
        wid = s * NC + c
        slab = s * ROWS_PER_TILE

        pltpu.sync_copy(zeros_hbm, acc.at[pl.ds(slab, ROWS_PER_TILE)])
        plsc.subcore_barrier()

        def start(i, b):
            chunk = wid + i * NW

            @pl.when(chunk < NCHUNK)
            def _():
                base = chunk * CHUNK
                pltpu.async_copy(dst_hbm.at[pl.ds(base, CHUNK)], didx[b],
                                 sem_i[b])
                pltpu.async_copy(msg_hbm.at[pl.ds(base, CHUNK)], rows[b],
                                 sem_m[b])

        def finish(i, b):
            chunk = wid + i * NW

            @pl.when(chunk < NCHUNK)
            def _():
                base = chunk * CHUNK
                pltpu.make_async_copy(dst_hbm.at[pl.ds(base, CHUNK)], didx[b],
                                      sem_i[b]).wait()
                pltpu.make_async_copy(msg_hbm.at[pl.ds(base, CHUNK)], rows[b],
                                      sem_m[b]).wait()
                pltpu.sync_copy(rows[b], acc.at[didx[b]], add=True)

        start(0, 0)

        @pl.loop(0, ITERS + 1, step=2)
        def _loop(i0):
            start(i0 + 1, 1)
            finish(i0, 0)
            start(i0 + 2, 0)
            finish(i0 + 1, 1)

        plsc.subcore_barrier()
        pltpu.sync_copy(acc.at[pl.ds(slab, ROWS_PER_TILE)],
                        out_hbm.at[c, pl.ds(slab, ROWS_PER_TILE)])

    return scatter_add


def _mlp_body(diff_ref, ef_ref, w1dT_ref, w1eT_ref, b1_ref,
              w2mT_ref, b2m_ref, w2aT_ref, b2a_ref, out_ref):
    x = diff_ref[...].astype(jnp.bfloat16)
    ef = ef_ref[...]
    h = (jnp.dot(x, w1dT_ref[...], preferred_element_type=jnp.float32)
         + jnp.dot(ef, w1eT_ref[...], preferred_element_type=jnp.float32)
         + b1_ref[...])
    h = jnp.maximum(h, 0.0).astype(jnp.bfloat16)
    hm = h[:, :MSG]
    ha = h[:, MSG:]
    msg = jnp.dot(hm, w2mT_ref[...], preferred_element_type=jnp.float32) + b2m_ref[...]
    att = jax.nn.sigmoid(
        jnp.dot(ha, w2aT_ref[...], preferred_element_type=jnp.float32)
        + b2a_ref[...])
    out_ref[...] = msg * att


def _run_mlp(diff, edge_feat, w1dT, w1eT, b1, w2mT, b2m, w2aT, b2a):
    BE = 3200
    grid = (EH // BE,)
    full = lambda shape: pl.BlockSpec(shape, lambda i: (0, 0))
    return pl.pallas_call(
        _mlp_body,
        grid=grid,
        in_specs=[
            pl.BlockSpec((BE, D), lambda i: (i, 0)),
            pl.BlockSpec((BE, DE), lambda i: (i, 0)),
            full((D, 2 * MSG)),
            full((DE, 2 * MSG)),
            full((1, 2 * MSG)),
            full((MSG, MSG)),
            full((1, MSG)),
            full((MSG, MSG)),
            full((1, MSG)),
        ],
        out_specs=pl.BlockSpec((BE, MSG), lambda i: (i, 0)),
        out_shape=jax.ShapeDtypeStruct((EH, MSG), jnp.float32),
        compiler_params=pltpu.CompilerParams(
            dimension_semantics=("arbitrary",),
        ),
    )(diff, edge_feat, w1dT, w1eT, b1, w2mT, b2m, w2aT, b2a)


def _gru_body(p0_ref, p1_ref, p2_ref, p3_ref, state_ref, wihT_ref, whhT_ref,
              bih_ref, bhh_ref, out_ref):
    m = (p0_ref[...] + p1_ref[...]) + (p2_ref[...] + p3_ref[...])
    state = state_ref[...]
    gi = m @ wihT_ref[...] + bih_ref[...]
    gh = state @ whhT_ref[...] + bhh_ref[...]
    r = jax.nn.sigmoid(gi[:, :D] + gh[:, :D])
    z = jax.nn.sigmoid(gi[:, D:2 * D] + gh[:, D:2 * D])
    n = jnp.tanh(gi[:, 2 * D:] + r * gh[:, 2 * D:])
    out_ref[...] = (1.0 - z) * n + z * state


def _run_gru(p0, p1, p2, p3, state, wihT, whhT, bih, bhh):
    BN = 1000
    grid = (N // BN,)
    full = lambda shape: pl.BlockSpec(shape, lambda i: (0, 0))
    return pl.pallas_call(
        _gru_body,
        grid=grid,
        in_specs=[
            pl.BlockSpec((BN, MSG), lambda i: (i, 0)),
            pl.BlockSpec((BN, MSG), lambda i: (i, 0)),
            pl.BlockSpec((BN, MSG), lambda i: (i, 0)),
            pl.BlockSpec((BN, MSG), lambda i: (i, 0)),
            pl.BlockSpec((BN, D), lambda i: (i, 0)),
            full((MSG, 3 * D)),
            full((D, 3 * D)),
            full((1, 3 * D)),
            full((1, 3 * D)),
        ],
        out_specs=pl.BlockSpec((BN, D), lambda i: (i, 0)),
        out_shape=jax.ShapeDtypeStruct((N, D), jnp.float32),
        compiler_params=pltpu.CompilerParams(
            dimension_semantics=("arbitrary",),
        ),
    )(p0, p1, p2, p3, state, wihT, whhT, bih, bhh)


def kernel(node_feat, edge, edge_feat, msg_W1, msg_b1, msg_W2, msg_b2,
           att_W1, att_b1, att_W2, att_b2, gru_Wih, gru_Whh, gru_bih, gru_bhh):
    src = edge[:, 0].astype(jnp.int32)
    dst = edge[:, 1].astype(jnp.int32)
    state = node_feat.astype(jnp.float32)

    w1dT = jnp.concatenate([msg_W1[:, :D].T, att_W1[:, :D].T], axis=1)
    w1eT = jnp.concatenate([msg_W1[:, D:].T, att_W1[:, D:].T], axis=1)
    b1 = jnp.concatenate([msg_b1, att_b1]).reshape(1, 2 * MSG)
    ef_bf = edge_feat.astype(jnp.bfloat16)
    mlp_w = (w1dT.astype(jnp.bfloat16), w1eT.astype(jnp.bfloat16), b1,
             msg_W2.T.astype(jnp.bfloat16), msg_b2.reshape(1, MSG),
             att_W2.T.astype(jnp.bfloat16), att_b2.reshape(1, MSG))
    zeros = jnp.zeros((ROWS_PER_TILE, D), jnp.float32)

    gather = _make_diff_gather()
    scatter = _make_scatter_add()

    halves = []
    for h in range(2):
        lo = h * EH
        s_h = src[lo:lo + EH]
        d_h = dst[lo:lo + EH]
        src3d = jnp.pad(s_h, (0, EPAD - EH)).reshape(NW, ITERS, CHUNK)
        dst3d = jnp.pad(d_h, (0, EPAD - EH)).reshape(NW, ITERS, CHUNK)
        halves.append((src3d, dst3d, d_h))

    diff0 = gather(state, halves[0][0], halves[0][1])
    diff1 = gather(state, halves[1][0], halves[1][1])
    msg0 = _run_mlp(diff0, ef_bf[:EH], *mlp_w)
    p0 = scatter(msg0, halves[0][2], zeros)
    msg1 = _run_mlp(diff1, ef_bf[EH:], *mlp_w)
    p1 = scatter(msg1, halves[1][2], zeros)

    return _run_gru(p0[0, :N], p0[1, :N], p1[0, :N], p1[1, :N], state,
                    gru_Wih.T, gru_Whh.T,
                    gru_bih.reshape(1, 3 * D), gru_bhh.reshape(1, 3 * D))


# speedup vs baseline: 1.2499x; 1.1983x over previous
import functools

import jax
import jax.numpy as jnp
from jax import lax
from jax.experimental import pallas as pl
from jax.experimental.pallas import tpu as pltpu
from jax.experimental.pallas import tpu_sc as plsc

N = 10000
E = 320000
D = 128
DE = 16
MSG = 128

NC = 2
NS = 16
NW = NC * NS

CHUNK = 128
EH = E // 2
NCHUNK = EH // CHUNK
ITERS = (NCHUNK + NW - 1) // NW
EPAD = NW * ITERS * CHUNK

NPAD = 10240
ROWS_PER_TILE = NPAD // NS

_sc_mesh = lambda: plsc.VectorSubcoreMesh(core_axis_name="c", subcore_axis_name="s")


def _make_diff_gather():
    @functools.partial(
        pl.kernel,
        out_type=jax.ShapeDtypeStruct((EH, D), jnp.float32),
        mesh=_sc_mesh(),
        scratch_types=[
            pltpu.VMEM((ITERS, CHUNK), jnp.int32),
            pltpu.VMEM((ITERS, CHUNK), jnp.int32),
            [pltpu.VMEM((CHUNK, D), jnp.float32)] * 2,
            [pltpu.VMEM((CHUNK, D), jnp.float32)] * 2,
            [pltpu.VMEM((CHUNK, D), jnp.float32)] * 2,
            [pltpu.SemaphoreType.DMA] * 2,
            [pltpu.SemaphoreType.DMA] * 2,
            [pltpu.SemaphoreType.DMA] * 2,
        ],
    )
    def diff_gather(state_hbm, src_hbm, dst_hbm, out_hbm,
                    sidx, didx, rows_a, rows_b, rows_o,
                    sem_a, sem_b, sem_o):
        c = lax.axis_index("c")
        s = lax.axis_index("s")
        wid = s * NC + c

        pltpu.sync_copy(src_hbm.at[wid], sidx)
        pltpu.sync_copy(dst_hbm.at[wid], didx)

        def start(i, b):
            chunk = wid * ITERS + i

            @pl.when(jnp.logical_and(i < ITERS, chunk < NCHUNK))
            def _():
                pltpu.async_copy(state_hbm.at[sidx.at[i]], rows_a[b],
                                 sem_a[b])
                pltpu.async_copy(state_hbm.at[didx.at[i]], rows_b[b],
                                 sem_b[b])

        def finish(i, b):
            chunk = wid * ITERS + i

            @pl.when(jnp.logical_and(i < ITERS, chunk < NCHUNK))
            def _():
                base = chunk * CHUNK
                pltpu.make_async_copy(state_hbm.at[sidx.at[i]], rows_a[b],
                                      sem_a[b]).wait()
                pltpu.make_async_copy(state_hbm.at[didx.at[i]], rows_b[b],
                                      sem_b[b]).wait()

                @pl.when(i >= 2)
                def _wait_prev_store():
                    pltpu.make_async_copy(
                        rows_o[b], out_hbm.at[pl.ds(0, CHUNK)], sem_o[b]
                    ).wait()

                def sub_row(r, carry2):
                    def sub_vec(j, carry3):
                        off = j * 16
                        rows_o[b][r, pl.ds(off, 16)] = (
                            rows_a[b][r, pl.ds(off, 16)]
                            - rows_b[b][r, pl.ds(off, 16)]
                        )
                        return carry3
                    return lax.fori_loop(0, D // 16, sub_vec, carry2,
                                         unroll=8)

                lax.fori_loop(0, CHUNK, sub_row, 0)
                pltpu.async_copy(rows_o[b], out_hbm.at[pl.ds(base, CHUNK)],
                                 sem_o[b])

        start(0, 0)

        @pl.loop(0, ITERS + 1, step=2)
        def _loop(i0):
            start(i0 + 1, 1)
            finish(i0, 0)
            start(i0 + 2, 0)
            finish(i0 + 1, 1)

        for b in range(2):
            pltpu.make_async_copy(rows_o[b], out_hbm.at[pl.ds(0, CHUNK)],
                                  sem_o[b]).wait()

    return diff_gather


def _make_scatter_add():
    @functools.partial(
        pl.kernel,
        out_type=jax.ShapeDtypeStruct((NC, NPAD, D), jnp.float32),
        mesh=_sc_mesh(),
        scratch_types=[
            [pltpu.VMEM((CHUNK,), jnp.int32)] * 2,
            [pltpu.VMEM((CHUNK, D), jnp.float32)] * 2,
            [pltpu.SemaphoreType.DMA] * 2,
            [pltpu.SemaphoreType.DMA] * 2,
            pltpu.VMEM_SHARED((NPAD, D), jnp.float32),
        ],
    )
    def scatter_add(msg_hbm, dst_hbm, zeros_hbm, out_hbm,
                    didx, rows, sem_i, sem_m, acc):
        c = lax.axis_index("c")
        s = lax.axis_index("s")
        wid = s * NC + c
        slab = s * ROWS_PER_TILE

        pltpu.sync_copy(zeros_hbm, acc.at[pl.ds(slab, ROWS_PER_TILE)])
        plsc.subcore_barrier()

        def start(i, b):
            chunk = wid + i * NW

            @pl.when(chunk < NCHUNK)
            def _():
                base = chunk * CHUNK
                pltpu.async_copy(dst_hbm.at[pl.ds(base, CHUNK)], didx[b],
                                 sem_i[b])
                pltpu.async_copy(msg_hbm.at[pl.ds(base, CHUNK)], rows[b],
                                 sem_m[b])

        def finish(i, b):
            chunk = wid + i * NW

            @pl.when(chunk < NCHUNK)
            def _():
                base = chunk * CHUNK
                pltpu.make_async_copy(dst_hbm.at[pl.ds(base, CHUNK)], didx[b],
                                      sem_i[b]).wait()
                pltpu.make_async_copy(msg_hbm.at[pl.ds(base, CHUNK)], rows[b],
                                      sem_m[b]).wait()
                pltpu.sync_copy(rows[b], acc.at[didx[b]], add=True)

        start(0, 0)

        @pl.loop(0, ITERS + 1, step=2)
        def _loop(i0):
            start(i0 + 1, 1)
            finish(i0, 0)
            start(i0 + 2, 0)
            finish(i0 + 1, 1)

        plsc.subcore_barrier()
        pltpu.sync_copy(acc.at[pl.ds(slab, ROWS_PER_TILE)],
                        out_hbm.at[c, pl.ds(slab, ROWS_PER_TILE)])

    return scatter_add


def _mlp_body(diff_ref, ef_ref, w1dT_ref, w1eT_ref, b1_ref,
              w2mT_ref, b2m_ref, w2aT_ref, b2a_ref, out_ref):
    x = diff_ref[...].astype(jnp.bfloat16)
    ef_t = ef_ref[...]
    h = (jnp.dot(x, w1dT_ref[...], preferred_element_type=jnp.float32)
         + lax.dot_general(ef_t, w1eT_ref[...],
                           (((0,), (0,)), ((), ())),
                           preferred_element_type=jnp.float32)
         + b1_ref[...])
    h = jnp.maximum(h, 0.0).astype(jnp.bfloat16)
    hm = h[:, :MSG]
    ha = h[:, MSG:]
    msg = jnp.dot(hm, w2mT_ref[...], preferred_element_type=jnp.float32) + b2m_ref[...]
    att = jax.nn.sigmoid(
        jnp.dot(ha, w2aT_ref[...], preferred_element_type=jnp.float32)
        + b2a_ref[...])
    out_ref[...] = msg * att


def _run_mlp(diff, edge_feat, w1dT, w1eT, b1, w2mT, b2m, w2aT, b2a):
    BE = 3200
    grid = (EH // BE,)
    full = lambda shape: pl.BlockSpec(shape, lambda i: (0, 0))
    return pl.pallas_call(
        _mlp_body,
        grid=grid,
        in_specs=[
            pl.BlockSpec((BE, D), lambda i: (i, 0)),
            pl.BlockSpec((DE, BE), lambda i: (0, i)),
            full((D, 2 * MSG)),
            full((DE, 2 * MSG)),
            full((1, 2 * MSG)),
            full((MSG, MSG)),
            full((1, MSG)),
            full((MSG, MSG)),
            full((1, MSG)),
        ],
        out_specs=pl.BlockSpec((BE, MSG), lambda i: (i, 0)),
        out_shape=jax.ShapeDtypeStruct((EH, MSG), jnp.float32),
        compiler_params=pltpu.CompilerParams(
            dimension_semantics=("arbitrary",),
        ),
    )(diff, edge_feat, w1dT, w1eT, b1, w2mT, b2m, w2aT, b2a)


def _gru_body(p0_ref, p1_ref, p2_ref, p3_ref, state_ref, wihT_ref, whhT_ref,
              bih_ref, bhh_ref, out_ref):
    m = (p0_ref[...] + p1_ref[...]) + (p2_ref[...] + p3_ref[...])
    state = state_ref[...]
    gi = m @ wihT_ref[...] + bih_ref[...]
    gh = state @ whhT_ref[...] + bhh_ref[...]
    r = jax.nn.sigmoid(gi[:, :D] + gh[:, :D])
    z = jax.nn.sigmoid(gi[:, D:2 * D] + gh[:, D:2 * D])
    n = jnp.tanh(gi[:, 2 * D:] + r * gh[:, 2 * D:])
    out_ref[...] = (1.0 - z) * n + z * state


def _run_gru(p0, p1, p2, p3, state, wihT, whhT, bih, bhh):
    BN = 1000
    grid = (N // BN,)
    full = lambda shape: pl.BlockSpec(shape, lambda i: (0, 0))
    return pl.pallas_call(
        _gru_body,
        grid=grid,
        in_specs=[
            pl.BlockSpec((BN, MSG), lambda i: (i, 0)),
            pl.BlockSpec((BN, MSG), lambda i: (i, 0)),
            pl.BlockSpec((BN, MSG), lambda i: (i, 0)),
            pl.BlockSpec((BN, MSG), lambda i: (i, 0)),
            pl.BlockSpec((BN, D), lambda i: (i, 0)),
            full((MSG, 3 * D)),
            full((D, 3 * D)),
            full((1, 3 * D)),
            full((1, 3 * D)),
        ],
        out_specs=pl.BlockSpec((BN, D), lambda i: (i, 0)),
        out_shape=jax.ShapeDtypeStruct((N, D), jnp.float32),
        compiler_params=pltpu.CompilerParams(
            dimension_semantics=("arbitrary",),
        ),
    )(p0, p1, p2, p3, state, wihT, whhT, bih, bhh)


def kernel(node_feat, edge, edge_feat, msg_W1, msg_b1, msg_W2, msg_b2,
           att_W1, att_b1, att_W2, att_b2, gru_Wih, gru_Whh, gru_bih, gru_bhh):
    src = edge[:, 0].astype(jnp.int32)
    dst = edge[:, 1].astype(jnp.int32)
    state = node_feat.astype(jnp.float32)

    w1dT = jnp.concatenate([msg_W1[:, :D].T, att_W1[:, :D].T], axis=1)
    w1eT = jnp.concatenate([msg_W1[:, D:].T, att_W1[:, D:].T], axis=1)
    b1 = jnp.concatenate([msg_b1, att_b1]).reshape(1, 2 * MSG)
    ef_t = edge_feat.T.astype(jnp.bfloat16)
    mlp_w = (w1dT.astype(jnp.bfloat16), w1eT.astype(jnp.bfloat16), b1,
             msg_W2.T.astype(jnp.bfloat16), msg_b2.reshape(1, MSG),
             att_W2.T.astype(jnp.bfloat16), att_b2.reshape(1, MSG))
    zeros = jnp.zeros((ROWS_PER_TILE, D), jnp.float32)

    gather = _make_diff_gather()
    scatter = _make_scatter_add()

    halves = []
    for h in range(2):
        lo = h * EH
        s_h = src[lo:lo + EH]
        d_h = dst[lo:lo + EH]
        src3d = jnp.pad(s_h, (0, EPAD - EH)).reshape(NW, ITERS, CHUNK)
        dst3d = jnp.pad(d_h, (0, EPAD - EH)).reshape(NW, ITERS, CHUNK)
        halves.append((src3d, dst3d, d_h))

    diff0 = gather(state, halves[0][0], halves[0][1])
    diff1 = gather(state, halves[1][0], halves[1][1])
    msg0 = _run_mlp(diff0, ef_t[:, :EH], *mlp_w)
    p0 = scatter(msg0, halves[0][2], zeros)
    msg1 = _run_mlp(diff1, ef_t[:, EH:], *mlp_w)
    p1 = scatter(msg1, halves[1][2], zeros)

    return _run_gru(p0[0, :N], p0[1, :N], p1[0, :N], p1[1, :N], state,
                    gru_Wih.T, gru_Whh.T,
                    gru_bih.reshape(1, 3 * D), gru_bhh.reshape(1, 3 * D))
